# baseline (speedup 1.0000x reference)
import functools

import jax
import jax.numpy as jnp
from jax import lax
from jax.experimental import pallas as pl
from jax.experimental.pallas import tpu as pltpu
from jax.experimental.pallas import tpu_sc as plsc

N_TOK = 8192
D_MODEL = 1024
N_EXP = 8
D_FF = 4096
EPAD = 128

RB = 512
T = 512
M_BLK = N_TOK // T
NT = M_BLK + N_EXP - 1


def _sublane_spread(v):
    sel = (lax.broadcasted_iota(jnp.int32, (8, EPAD), 0)
           == lax.broadcasted_iota(jnp.int32, (8, EPAD), 1))
    return jnp.sum(jnp.where(sel, jnp.broadcast_to(v, (8, EPAD)), 0),
                   axis=1, keepdims=True)


def _emit_meta(counts_f, meta_ref):
    tri_incl = (lax.broadcasted_iota(jnp.int32, (EPAD, EPAD), 0)
                <= lax.broadcasted_iota(jnp.int32, (EPAD, EPAD), 1))
    ends_f = jax.lax.dot_general(counts_f, tri_incl.astype(jnp.float32),
                                 (((1,), (0,)), ((), ())),
                                 preferred_element_type=jnp.float32,
                                 precision=jax.lax.Precision.HIGHEST)
    counts_i = counts_f.astype(jnp.int32)
    ends_i = ends_f.astype(jnp.int32)
    starts_i = ends_i - counts_i
    nz = counts_i > 0
    firstb = jnp.where(nz, lax.shift_right_arithmetic(starts_i, 9), 0)
    lastb = jnp.where(nz, lax.shift_right_arithmetic(ends_i - 1, 9), -1)
    nblk = jnp.where(nz, lastb - firstb + 1, 0)
    offs_f = jax.lax.dot_general(nblk.astype(jnp.float32),
                                 tri_incl.astype(jnp.float32),
                                 (((1,), (0,)), ((), ())),
                                 preferred_element_type=jnp.float32,
                                 precision=jax.lax.Precision.HIGHEST)
    offs_i = offs_f.astype(jnp.int32)
    offsx_i = offs_i - nblk
    ti = lax.broadcasted_iota(jnp.int32, (8, EPAD), 1)
    ind = ((ti >= _sublane_spread(offsx_i)) & (ti < _sublane_spread(offs_i))
           ).astype(jnp.int32)
    erow = lax.broadcasted_iota(jnp.int32, (8, EPAD), 0)
    gv = jnp.sum(ind * erow, axis=0, keepdims=True)
    validv = jnp.sum(ind, axis=0, keepdims=True) > 0
    localv = jnp.sum(ind * (ti - _sublane_spread(offsx_i)), axis=0,
                     keepdims=True)
    mv = jnp.sum(ind * _sublane_spread(firstb), axis=0, keepdims=True) + localv
    lov = jnp.sum(ind * _sublane_spread(starts_i), axis=0, keepdims=True)
    hiv = jnp.sum(ind * _sublane_spread(ends_i), axis=0, keepdims=True)
    g_last = jnp.max(jnp.where(validv, gv, 0))
    mv = jnp.where(validv, mv, M_BLK - 1)
    gv = jnp.where(validv, gv, g_last)
    lov = jnp.where(validv, lov, 0)
    hiv = jnp.where(validv, hiv, 0)
    prev_m = jnp.concatenate(
        [jnp.full((1, 1), -1, jnp.int32), mv[:, :EPAD - 1]], axis=1)
    firstv = (mv != prev_m).astype(jnp.int32)
    meta_ref[0:1, :] = mv
    meta_ref[1:2, :] = gv
    meta_ref[2:3, :] = lov
    meta_ref[3:4, :] = hiv
    meta_ref[4:5, :] = firstv
    meta_ref[5:6, :] = starts_i
    meta_ref[6:7, :] = jnp.zeros((1, EPAD), jnp.int32)
    meta_ref[7:8, :] = jnp.zeros((1, EPAD), jnp.int32)


def _router_body(x_ref, w_ref, b_ref, xs_ref, routes_ref, rank_ref, meta_ref,
                 carry_ref):
    i = pl.program_id(0)

    @pl.when(i == 0)
    def _():
        carry_ref[...] = jnp.zeros_like(carry_ref)

    xb = x_ref[...]
    logits = jax.lax.dot_general(
        xb, w_ref[...], (((1,), (0,)), ((), ())),
        preferred_element_type=jnp.float32)
    logits = logits + b_ref[...]
    lmax = jnp.max(logits, axis=1, keepdims=True)
    col = lax.broadcasted_iota(jnp.int32, (RB, EPAD), 1)
    routes = jnp.min(jnp.where(logits == lmax, col, EPAD), axis=1)
    sumexp = jnp.sum(jnp.exp(logits - lmax), axis=1, keepdims=True)
    pmax = 1.0 / sumexp
    xs_ref[...] = xb * pmax

    onehot = (col == routes[:, None]).astype(jnp.float32)
    rowi = lax.broadcasted_iota(jnp.int32, (RB, RB), 0)
    coli = lax.broadcasted_iota(jnp.int32, (RB, RB), 1)
    ltri = (rowi > coli).astype(jnp.float32)
    cs = jax.lax.dot_general(ltri, onehot, (((1,), (0,)), ((), ())),
                             preferred_element_type=jnp.float32)
    carry = carry_ref[0:1, :]
    rank = jnp.sum(onehot * (cs + carry), axis=1)
    routes_ref[0, 0, :] = routes
    rank_ref[0, 0, :] = rank.astype(jnp.int32)
    carry_ref[0:1, :] = carry + jnp.sum(onehot, axis=0, keepdims=True)

    @pl.when(i == pl.num_programs(0) - 1)
    def _():
        _emit_meta(carry_ref[0:1, :], meta_ref)


def _router(x2d, w_pad, b_pad):
    nblk = N_TOK // RB
    return pl.pallas_call(
        _router_body,
        grid=(nblk,),
        in_specs=[
            pl.BlockSpec((RB, D_MODEL), lambda i: (i, 0)),
            pl.BlockSpec((D_MODEL, EPAD), lambda i: (0, 0)),
            pl.BlockSpec((1, EPAD), lambda i: (0, 0)),
        ],
        out_specs=[
            pl.BlockSpec((RB, D_MODEL), lambda i: (i, 0)),
            pl.BlockSpec((1, 1, RB), lambda i: (i, 0, 0)),
            pl.BlockSpec((1, 1, RB), lambda i: (i, 0, 0)),
            pl.BlockSpec((8, EPAD), lambda i: (0, 0)),
        ],
        out_shape=[
            jax.ShapeDtypeStruct((N_TOK, D_MODEL), jnp.float32),
            jax.ShapeDtypeStruct((nblk, 1, RB), jnp.int32),
            jax.ShapeDtypeStruct((nblk, 1, RB), jnp.int32),
            jax.ShapeDtypeStruct((8, EPAD), jnp.int32),
        ],
        scratch_shapes=[pltpu.VMEM((8, EPAD), jnp.float32)],
    )(x2d, w_pad, b_pad)


_SC_CHUNK = 32


def _make_dispatch():
    info = plsc.get_sparse_core_info()
    nc, ns = info.num_cores, info.num_subcores
    nw = nc * ns
    per_w = N_TOK // nw
    nchunk = per_w // _SC_CHUNK
    mesh = plsc.VectorSubcoreMesh(core_axis_name="c", subcore_axis_name="s")

    @functools.partial(
        pl.kernel,
        out_type=jax.ShapeDtypeStruct((N_TOK, D_MODEL), jnp.float32),
        mesh=mesh,
        compiler_params=pltpu.CompilerParams(needs_layout_passes=False),
        scratch_types=[
            pltpu.VMEM((16,), jnp.int32),
            pltpu.VMEM((per_w,), jnp.int32),
            pltpu.VMEM((per_w,), jnp.int32),
            [pltpu.VMEM((_SC_CHUNK,), jnp.int32) for _ in range(nchunk)],
            [pltpu.VMEM((_SC_CHUNK, D_MODEL), jnp.float32) for _ in range(2)],
            [pltpu.SemaphoreType.DMA for _ in range(2)],
            [pltpu.SemaphoreType.DMA for _ in range(2)],
        ],
    )
    def dispatch(xs_hbm, routes_hbm, rank_hbm, meta_hbm, out_hbm,
                 starts_v, routes_v, rank_v, pos_vs, bufs, rsem, wsem):
        wid = lax.axis_index("s") * nc + lax.axis_index("c")
        base = wid * per_w
        pltpu.sync_copy(meta_hbm.at[5, pl.ds(0, 16)], starts_v)
        pltpu.sync_copy(routes_hbm.at[pl.ds(base, per_w)], routes_v)
        pltpu.sync_copy(rank_hbm.at[pl.ds(base, per_w)], rank_v)
        for j in range(per_w // 16):
            r = routes_v[pl.ds(j * 16, 16)]
            s = plsc.load_gather(starts_v, [r])
            k = rank_v[pl.ds(j * 16, 16)]
            cidx, off = divmod(j * 16, _SC_CHUNK)
            pos_vs[cidx][pl.ds(off, 16)] = s + k
        reads = [None] * nchunk
        writes = [None] * nchunk

        def read(c, b):
            return pltpu.async_copy(
                xs_hbm.at[pl.ds(base + c * _SC_CHUNK, _SC_CHUNK)],
                bufs[b], rsem[b])

        reads[0] = read(0, 0)
        for c in range(nchunk):
            b = c & 1
            reads[c].wait()
            writes[c] = pltpu.async_copy(bufs[b], out_hbm.at[pos_vs[c]],
                                         wsem[b])
            if c + 1 < nchunk:
                if c >= 1:
                    writes[c - 1].wait()
                reads[c + 1] = read(c + 1, 1 - b)
        writes[nchunk - 2].wait()
        writes[nchunk - 1].wait()

    return dispatch


FH = D_FF // 2


def _ffn_body(meta_ref, x_ref, w1_ref, b1_ref, w2_ref, b2_ref, oin_ref, o_ref):
    f = pl.program_id(0)
    t = pl.program_id(1)
    m = meta_ref[0, t]
    lo = meta_ref[2, t]
    hi = meta_ref[3, t]
    first = meta_ref[4, t]

    @pl.when(hi > lo)
    def _():
        h = jax.lax.dot_general(x_ref[...], w1_ref[0], (((1,), (0,)), ((), ())),
                                preferred_element_type=jnp.float32)
        h = jnp.maximum(h + b1_ref[0], 0.0)
        y = jax.lax.dot_general(h, w2_ref[0], (((1,), (0,)), ((), ())),
                                preferred_element_type=jnp.float32)
        y = y + b2_ref[0] * (f == 0).astype(jnp.float32)
        rows = m * T + lax.broadcasted_iota(jnp.int32, (T, 1), 0)
        contrib = jnp.where((rows >= lo) & (rows < hi), y, 0.0)

        @pl.when((first == 1) & (f == 0))
        def _():
            o_ref[...] = contrib

        @pl.when((first == 1) & (f == 1))
        def _():
            o_ref[...] = oin_ref[...] + contrib

        @pl.when(first == 0)
        def _():
            o_ref[...] = o_ref[...] + contrib


def _ffn(meta, xsorted, w1, b1, w2, b2, acc):
    grid_spec = pltpu.PrefetchScalarGridSpec(
        num_scalar_prefetch=1,
        grid=(2, NT),
        in_specs=[
            pl.BlockSpec((T, D_MODEL), lambda f, t, mr: (mr[0, t], 0)),
            pl.BlockSpec((1, D_MODEL, FH), lambda f, t, mr: (mr[1, t], 0, f)),
            pl.BlockSpec((1, 1, FH), lambda f, t, mr: (mr[1, t], 0, f)),
            pl.BlockSpec((1, FH, D_MODEL), lambda f, t, mr: (mr[1, t], f, 0)),
            pl.BlockSpec((1, 1, D_MODEL), lambda f, t, mr: (mr[1, t], 0, 0)),
            pl.BlockSpec((T, D_MODEL), lambda f, t, mr: (mr[0, t], 0)),
        ],
        out_specs=pl.BlockSpec((T, D_MODEL), lambda f, t, mr: (mr[0, t], 0)),
    )
    return pl.pallas_call(
        _ffn_body,
        grid_spec=grid_spec,
        out_shape=jax.ShapeDtypeStruct((N_TOK, D_MODEL), jnp.float32),
        input_output_aliases={6: 0},
    )(meta, xsorted, w1, b1, w2, b2, acc)


@jax.jit
def kernel(x, w_switch, b_switch, W1, b1, W2, b2):
    bsz, seq, _ = x.shape
    x2d = x.reshape(N_TOK, D_MODEL)

    w_pad = jnp.zeros((D_MODEL, EPAD), jnp.float32).at[:, :N_EXP].set(w_switch.T)
    b_pad = jnp.full((1, EPAD), -1e30, jnp.float32).at[0, :N_EXP].set(b_switch)

    xs, routes3d, rank3d, meta = _router(x2d, w_pad, b_pad)
    routes = routes3d.reshape(N_TOK)
    rank = rank3d.reshape(N_TOK)

    xsorted = _make_dispatch()(xs, routes, rank, meta)

    out = _ffn(meta, xsorted,
               W1, b1.reshape(N_EXP, 1, D_FF),
               W2, b2.reshape(N_EXP, 1, D_MODEL), xs)
    return out.reshape(bsz, seq, D_MODEL)


# speedup vs baseline: 1.0010x; 1.0010x over previous
import functools

import jax
import jax.numpy as jnp
from jax import lax
from jax.experimental import pallas as pl
from jax.experimental.pallas import tpu as pltpu
from jax.experimental.pallas import tpu_sc as plsc

N_TOK = 8192
D_MODEL = 1024
N_EXP = 8
D_FF = 4096
EPAD = 128

RB = 512
T = 512
M_BLK = N_TOK // T
NT = M_BLK + N_EXP - 1


def _sublane_spread(v):
    sel = (lax.broadcasted_iota(jnp.int32, (8, EPAD), 0)
           == lax.broadcasted_iota(jnp.int32, (8, EPAD), 1))
    return jnp.sum(jnp.where(sel, jnp.broadcast_to(v, (8, EPAD)), 0),
                   axis=1, keepdims=True)


def _emit_meta(counts_f, meta_ref):
    tri_incl = (lax.broadcasted_iota(jnp.int32, (EPAD, EPAD), 0)
                <= lax.broadcasted_iota(jnp.int32, (EPAD, EPAD), 1))
    ends_f = jax.lax.dot_general(counts_f, tri_incl.astype(jnp.float32),
                                 (((1,), (0,)), ((), ())),
                                 preferred_element_type=jnp.float32,
                                 precision=jax.lax.Precision.HIGHEST)
    counts_i = counts_f.astype(jnp.int32)
    ends_i = ends_f.astype(jnp.int32)
    starts_i = ends_i - counts_i
    nz = counts_i > 0
    firstb = jnp.where(nz, lax.shift_right_arithmetic(starts_i, 9), 0)
    lastb = jnp.where(nz, lax.shift_right_arithmetic(ends_i - 1, 9), -1)
    nblk = jnp.where(nz, lastb - firstb + 1, 0)
    offs_f = jax.lax.dot_general(nblk.astype(jnp.float32),
                                 tri_incl.astype(jnp.float32),
                                 (((1,), (0,)), ((), ())),
                                 preferred_element_type=jnp.float32,
                                 precision=jax.lax.Precision.HIGHEST)
    offs_i = offs_f.astype(jnp.int32)
    offsx_i = offs_i - nblk
    ti = lax.broadcasted_iota(jnp.int32, (8, EPAD), 1)
    ind = ((ti >= _sublane_spread(offsx_i)) & (ti < _sublane_spread(offs_i))
           ).astype(jnp.int32)
    erow = lax.broadcasted_iota(jnp.int32, (8, EPAD), 0)
    gv = jnp.sum(ind * erow, axis=0, keepdims=True)
    validv = jnp.sum(ind, axis=0, keepdims=True) > 0
    localv = jnp.sum(ind * (ti - _sublane_spread(offsx_i)), axis=0,
                     keepdims=True)
    mv = jnp.sum(ind * _sublane_spread(firstb), axis=0, keepdims=True) + localv
    lov = jnp.sum(ind * _sublane_spread(starts_i), axis=0, keepdims=True)
    hiv = jnp.sum(ind * _sublane_spread(ends_i), axis=0, keepdims=True)
    g_last = jnp.max(jnp.where(validv, gv, 0))
    mv = jnp.where(validv, mv, M_BLK - 1)
    gv = jnp.where(validv, gv, g_last)
    lov = jnp.where(validv, lov, 0)
    hiv = jnp.where(validv, hiv, 0)
    prev_m = jnp.concatenate(
        [jnp.full((1, 1), -1, jnp.int32), mv[:, :EPAD - 1]], axis=1)
    firstv = (mv != prev_m).astype(jnp.int32)
    meta_ref[0:1, :] = mv
    meta_ref[1:2, :] = gv
    meta_ref[2:3, :] = lov
    meta_ref[3:4, :] = hiv
    meta_ref[4:5, :] = firstv
    meta_ref[5:6, :] = starts_i
    meta_ref[6:7, :] = jnp.zeros((1, EPAD), jnp.int32)
    meta_ref[7:8, :] = jnp.zeros((1, EPAD), jnp.int32)


def _router_body(x_ref, w_ref, b_ref, xs_ref, routes_ref, rank_ref, meta_ref,
                 carry_ref):
    i = pl.program_id(0)

    @pl.when(i == 0)
    def _():
        carry_ref[...] = jnp.zeros_like(carry_ref)

    xb = x_ref[...]
    logits = jax.lax.dot_general(
        xb, w_ref[...], (((1,), (0,)), ((), ())),
        preferred_element_type=jnp.float32)
    logits = logits + b_ref[...]
    lmax = jnp.max(logits, axis=1, keepdims=True)
    col = lax.broadcasted_iota(jnp.int32, (RB, EPAD), 1)
    routes = jnp.min(jnp.where(logits == lmax, col, EPAD), axis=1)
    sumexp = jnp.sum(jnp.exp(logits - lmax), axis=1, keepdims=True)
    pmax = 1.0 / sumexp
    xs_ref[...] = xb * pmax

    onehot = (col == routes[:, None]).astype(jnp.float32)
    rowi = lax.broadcasted_iota(jnp.int32, (RB, RB), 0)
    coli = lax.broadcasted_iota(jnp.int32, (RB, RB), 1)
    ltri = (rowi > coli).astype(jnp.float32)
    cs = jax.lax.dot_general(ltri, onehot, (((1,), (0,)), ((), ())),
                             preferred_element_type=jnp.float32)
    carry = carry_ref[0:1, :]
    rank = jnp.sum(onehot * (cs + carry), axis=1)
    routes_ref[0, 0, :] = routes
    rank_ref[0, 0, :] = rank.astype(jnp.int32)
    carry_ref[0:1, :] = carry + jnp.sum(onehot, axis=0, keepdims=True)

    @pl.when(i == pl.num_programs(0) - 1)
    def _():
        _emit_meta(carry_ref[0:1, :], meta_ref)


def _router(x2d, w_pad, b_pad):
    nblk = N_TOK // RB
    return pl.pallas_call(
        _router_body,
        grid=(nblk,),
        in_specs=[
            pl.BlockSpec((RB, D_MODEL), lambda i: (i, 0)),
            pl.BlockSpec((D_MODEL, EPAD), lambda i: (0, 0)),
            pl.BlockSpec((1, EPAD), lambda i: (0, 0)),
        ],
        out_specs=[
            pl.BlockSpec((RB, D_MODEL), lambda i: (i, 0)),
            pl.BlockSpec((1, 1, RB), lambda i: (i, 0, 0)),
            pl.BlockSpec((1, 1, RB), lambda i: (i, 0, 0)),
            pl.BlockSpec((8, EPAD), lambda i: (0, 0)),
        ],
        out_shape=[
            jax.ShapeDtypeStruct((N_TOK, D_MODEL), jnp.float32),
            jax.ShapeDtypeStruct((nblk, 1, RB), jnp.int32),
            jax.ShapeDtypeStruct((nblk, 1, RB), jnp.int32),
            jax.ShapeDtypeStruct((8, EPAD), jnp.int32),
        ],
        scratch_shapes=[pltpu.VMEM((8, EPAD), jnp.float32)],
    )(x2d, w_pad, b_pad)


_SC_CHUNK = 32


def _make_dispatch():
    info = plsc.get_sparse_core_info()
    nc, ns = info.num_cores, info.num_subcores
    nw = nc * ns
    per_w = N_TOK // nw
    nchunk = per_w // _SC_CHUNK
    mesh = plsc.VectorSubcoreMesh(core_axis_name="c", subcore_axis_name="s")

    @functools.partial(
        pl.kernel,
        out_type=jax.ShapeDtypeStruct((N_TOK, D_MODEL), jnp.float32),
        mesh=mesh,
        compiler_params=pltpu.CompilerParams(needs_layout_passes=False),
        scratch_types=[
            pltpu.VMEM((16,), jnp.int32),
            pltpu.VMEM((per_w,), jnp.int32),
            pltpu.VMEM((per_w,), jnp.int32),
            [pltpu.VMEM((_SC_CHUNK,), jnp.int32) for _ in range(nchunk)],
            [pltpu.VMEM((_SC_CHUNK, D_MODEL), jnp.float32) for _ in range(2)],
            [pltpu.SemaphoreType.DMA for _ in range(2)],
            [pltpu.SemaphoreType.DMA for _ in range(2)],
        ],
    )
    def dispatch(xs_hbm, routes_hbm, rank_hbm, meta_hbm, out_hbm,
                 starts_v, routes_v, rank_v, pos_vs, bufs, rsem, wsem):
        wid = lax.axis_index("s") * nc + lax.axis_index("c")
        base = wid * per_w
        pltpu.sync_copy(meta_hbm.at[5, pl.ds(0, 16)], starts_v)
        pltpu.sync_copy(routes_hbm.at[pl.ds(base, per_w)], routes_v)
        pltpu.sync_copy(rank_hbm.at[pl.ds(base, per_w)], rank_v)
        for j in range(per_w // 16):
            r = routes_v[pl.ds(j * 16, 16)]
            s = plsc.load_gather(starts_v, [r])
            k = rank_v[pl.ds(j * 16, 16)]
            cidx, off = divmod(j * 16, _SC_CHUNK)
            pos_vs[cidx][pl.ds(off, 16)] = s + k
        reads = [None] * nchunk
        writes = [None] * nchunk

        def read(c, b):
            return pltpu.async_copy(
                xs_hbm.at[pl.ds(base + c * _SC_CHUNK, _SC_CHUNK)],
                bufs[b], rsem[b])

        reads[0] = read(0, 0)
        for c in range(nchunk):
            b = c & 1
            reads[c].wait()
            writes[c] = pltpu.async_copy(bufs[b], out_hbm.at[pos_vs[c]],
                                         wsem[b])
            if c + 1 < nchunk:
                if c >= 1:
                    writes[c - 1].wait()
                reads[c + 1] = read(c + 1, 1 - b)
        writes[nchunk - 2].wait()
        writes[nchunk - 1].wait()

    return dispatch


FH = D_FF // 2


def _ffn_body(meta_ref, x_ref, w1_ref, b1_ref, w2_ref, b2_ref, oin_ref, o_ref):
    f = pl.program_id(0)
    t = pl.program_id(1)
    m = meta_ref[0, t]
    lo = meta_ref[2, t]
    hi = meta_ref[3, t]
    first = meta_ref[4, t]

    @pl.when(hi > lo)
    def _():
        xb = x_ref[...].astype(jnp.bfloat16)
        h = jax.lax.dot_general(xb, w1_ref[0], (((1,), (0,)), ((), ())),
                                preferred_element_type=jnp.float32)
        h = jnp.maximum(h + b1_ref[0], 0.0).astype(jnp.bfloat16)
        y = jax.lax.dot_general(h, w2_ref[0], (((1,), (0,)), ((), ())),
                                preferred_element_type=jnp.float32)
        y = y + b2_ref[0] * (f == 0).astype(jnp.float32)
        rows = m * T + lax.broadcasted_iota(jnp.int32, (T, 1), 0)
        contrib = jnp.where((rows >= lo) & (rows < hi), y, 0.0)

        @pl.when((first == 1) & (f == 0))
        def _():
            o_ref[...] = contrib

        @pl.when((first == 1) & (f == 1))
        def _():
            o_ref[...] = oin_ref[...] + contrib

        @pl.when(first == 0)
        def _():
            o_ref[...] = o_ref[...] + contrib


def _ffn(meta, xsorted, w1, b1, w2, b2, acc):
    grid_spec = pltpu.PrefetchScalarGridSpec(
        num_scalar_prefetch=1,
        grid=(2, NT),
        in_specs=[
            pl.BlockSpec((T, D_MODEL), lambda f, t, mr: (mr[0, t], 0)),
            pl.BlockSpec((1, D_MODEL, FH), lambda f, t, mr: (mr[1, t], 0, f)),
            pl.BlockSpec((1, 1, FH), lambda f, t, mr: (mr[1, t], 0, f)),
            pl.BlockSpec((1, FH, D_MODEL), lambda f, t, mr: (mr[1, t], f, 0)),
            pl.BlockSpec((1, 1, D_MODEL), lambda f, t, mr: (mr[1, t], 0, 0)),
            pl.BlockSpec((T, D_MODEL), lambda f, t, mr: (mr[0, t], 0)),
        ],
        out_specs=pl.BlockSpec((T, D_MODEL), lambda f, t, mr: (mr[0, t], 0)),
    )
    return pl.pallas_call(
        _ffn_body,
        grid_spec=grid_spec,
        out_shape=jax.ShapeDtypeStruct((N_TOK, D_MODEL), jnp.float32),
        input_output_aliases={6: 0},
    )(meta, xsorted, w1, b1, w2, b2, acc)


@jax.jit
def kernel(x, w_switch, b_switch, W1, b1, W2, b2):
    bsz, seq, _ = x.shape
    x2d = x.reshape(N_TOK, D_MODEL)

    w_pad = jnp.zeros((D_MODEL, EPAD), jnp.float32).at[:, :N_EXP].set(w_switch.T)
    b_pad = jnp.full((1, EPAD), -1e30, jnp.float32).at[0, :N_EXP].set(b_switch)

    xs, routes3d, rank3d, meta = _router(x2d, w_pad, b_pad)
    routes = routes3d.reshape(N_TOK)
    rank = rank3d.reshape(N_TOK)

    xsorted = _make_dispatch()(xs, routes, rank, meta)

    out = _ffn(meta, xsorted,
               W1, b1.reshape(N_EXP, 1, D_FF),
               W2, b2.reshape(N_EXP, 1, D_MODEL), xs)
    return out.reshape(bsz, seq, D_MODEL)
